# baseline (speedup 1.0000x reference)
import functools

import jax
import jax.numpy as jnp
from jax import lax
from jax.experimental import pallas as pl
from jax.experimental.pallas import tpu as pltpu
from jax.experimental.pallas import tpu_sc as plsc

_NUM_CODEBOOKS = 4
_SUB_VOCAB = 250000
_HIDDEN = 64
_BATCH = 16384

_NW = 32
_TOTAL = _NUM_CODEBOOKS * _BATCH
_PER_W = _TOTAL // _NW
_NBLK = 4
_WB = _PER_W // _NBLK
_G = 128
_GROUPS = _WB // _G
_TOTAL_ROWS_W = _NUM_CODEBOOKS * _SUB_VOCAB // 2

_mesh = plsc.VectorSubcoreMesh(core_axis_name="c", subcore_axis_name="s")


@functools.partial(
    pl.kernel,
    out_type=jax.ShapeDtypeStruct((_TOTAL, 2 * _HIDDEN), jnp.float32),
    mesh=_mesh,
    scratch_types=[
        pltpu.VMEM((8, _G), jnp.int32),
        pltpu.VMEM((_WB, 2 * _HIDDEN), jnp.float32),
        pltpu.SemaphoreType.DMA,
    ],
)
def _gather_wide(table_hbm, idx_hbm, out_hbm, idx_v, rows_v, sem):
    wid = lax.axis_index("s") * 2 + lax.axis_index("c")
    for k in range(_NBLK):
        row = wid * _NBLK + k
        pltpu.sync_copy(idx_hbm.at[row], idx_v)
        copies = []
        for g in range(_GROUPS):
            dst = rows_v.at[pl.ds(g * _G, _G)]
            copies.append(pltpu.async_copy(table_hbm.at[idx_v.at[g]], dst, sem))
        for cpy in copies:
            cpy.wait()
        base = wid * _PER_W + k * _WB
        pltpu.sync_copy(rows_v, out_hbm.at[pl.ds(base, _WB)])


_TBLK = 512


def _fold_body(g_ref, p_ref, o_ref):
    g = g_ref[...]
    p = p_ref[...]
    sel = jnp.where(p[:, :, None] == 1, g[:, :, _HIDDEN:], g[:, :, :_HIDDEN])
    o_ref[...] = jnp.sum(sel, axis=0)


_fold = pl.pallas_call(
    _fold_body,
    out_shape=jax.ShapeDtypeStruct((_BATCH, _HIDDEN), jnp.float32),
    grid=(_BATCH // _TBLK,),
    in_specs=[
        pl.BlockSpec((_NUM_CODEBOOKS, _TBLK, 2 * _HIDDEN), lambda i: (0, i, 0)),
        pl.BlockSpec((_NUM_CODEBOOKS, _TBLK), lambda i: (0, i)),
    ],
    out_specs=pl.BlockSpec((_TBLK, _HIDDEN), lambda i: (i, 0)),
)


def kernel(indices, codebooks):
    table = codebooks.reshape(_TOTAL_ROWS_W, 2 * _HIDDEN)
    offs = (jnp.arange(_NUM_CODEBOOKS, dtype=jnp.int32) * _SUB_VOCAB)[None, :]
    flat = indices + offs
    parity = (flat & 1).T
    wide = (flat >> 1).T.reshape(_TOTAL)
    idx_arr = jnp.pad(wide.reshape(_NW * _NBLK, _GROUPS, _G),
                      ((0, 0), (0, 8 - _GROUPS), (0, 0)))
    gathered = _gather_wide(table, idx_arr)
    g3 = gathered.reshape(_NUM_CODEBOOKS, _BATCH, 2 * _HIDDEN)
    return _fold(g3, parity)


# speedup vs baseline: 1.0589x; 1.0589x over previous
import functools

import jax
import jax.numpy as jnp
from jax import lax
from jax.experimental import pallas as pl
from jax.experimental.pallas import tpu as pltpu
from jax.experimental.pallas import tpu_sc as plsc

_NUM_CODEBOOKS = 4
_SUB_VOCAB = 250000
_HIDDEN = 64
_BATCH = 16384

_NW = 32
_NS = 16
_TOK_PER_W = _BATCH // _NW
_NBLK = 2
_WB = _TOK_PER_W // _NBLK
_G = 128
_GROUPS = _NUM_CODEBOOKS * _WB // _G
_C0_GROUPS = _WB // _G
_ADD_GROUPS = _GROUPS - _C0_GROUPS

_mesh = plsc.VectorSubcoreMesh(core_axis_name="c", subcore_axis_name="s")


@functools.partial(
    pl.kernel,
    out_type=jax.ShapeDtypeStruct((_BATCH, _HIDDEN), jnp.float32),
    mesh=_mesh,
    compiler_params=pltpu.CompilerParams(use_tc_tiling_on_sc=False),
    scratch_types=[
        pltpu.VMEM((_GROUPS, _G), jnp.int32),
        pltpu.VMEM((_GROUPS * _G, _HIDDEN), jnp.float32),
        pltpu.VMEM_SHARED((_NS * _WB, _HIDDEN), jnp.float32),
    ]
    + [pltpu.VMEM((_G,), jnp.int32) for _ in range(_ADD_GROUPS)]
    + [pltpu.SemaphoreType.DMA],
)
def _embed(table_hbm, idx_hbm, pos_hbm, out_hbm, idx_v, rows_v, acc_sh,
           p0, p1, p2, p3, p4, p5, sem):
    sid = lax.axis_index("s")
    wid = sid * 2 + lax.axis_index("c")
    pos_refs = (p0, p1, p2, p3, p4, p5)
    for j, p in enumerate(pos_refs):
        pltpu.sync_copy(pos_hbm.at[sid, j], p)
    for k in range(_NBLK):
        row = wid * _NBLK + k
        pltpu.sync_copy(idx_hbm.at[row], idx_v)
        copies = []
        for g in range(_GROUPS):
            dst = rows_v.at[pl.ds(g * _G, _G)]
            copies.append(pltpu.async_copy(table_hbm.at[idx_v.at[g]], dst, sem))
        for cpy in copies:
            cpy.wait()
        pltpu.sync_copy(rows_v.at[pl.ds(0, _WB)],
                        acc_sh.at[pl.ds(sid * _WB, _WB)])
        for j, p in enumerate(pos_refs):
            pltpu.sync_copy(rows_v.at[pl.ds(_WB + j * _G, _G)],
                            acc_sh.at[p], add=True)
        base = wid * _TOK_PER_W + k * _WB
        pltpu.sync_copy(acc_sh.at[pl.ds(sid * _WB, _WB)],
                        out_hbm.at[pl.ds(base, _WB)])


def kernel(indices, codebooks):
    table = codebooks.reshape(_NUM_CODEBOOKS * _SUB_VOCAB, _HIDDEN)
    offs = (jnp.arange(_NUM_CODEBOOKS, dtype=jnp.int32) * _SUB_VOCAB)[None, :]
    flat = indices + offs
    idx_arr = (flat.reshape(_NW, _NBLK, _WB, _NUM_CODEBOOKS)
               .transpose(0, 1, 3, 2)
               .reshape(_NW * _NBLK, _GROUPS, _G))
    sid_off = (jnp.arange(_NS, dtype=jnp.int32) * _WB)[:, None, None]
    grp_off = ((jnp.arange(_ADD_GROUPS, dtype=jnp.int32) % _C0_GROUPS)
               * _G)[None, :, None]
    lane = jnp.arange(_G, dtype=jnp.int32)[None, None, :]
    pos = sid_off + grp_off + lane
    return _embed(table, idx_arr, pos)


# speedup vs baseline: 1.0606x; 1.0016x over previous
import functools

import jax
import jax.numpy as jnp
from jax import lax
from jax.experimental import pallas as pl
from jax.experimental.pallas import tpu as pltpu
from jax.experimental.pallas import tpu_sc as plsc

_NUM_CODEBOOKS = 4
_SUB_VOCAB = 250000
_HIDDEN = 64
_BATCH = 16384

_NW = 32
_NS = 16
_TOK_PER_W = _BATCH // _NW
_NBLK = 2
_WB = _TOK_PER_W // _NBLK
_G = 128
_GROUPS = _NUM_CODEBOOKS * _WB // _G
_C0_GROUPS = _WB // _G
_ADD_GROUPS = _GROUPS - _C0_GROUPS

_mesh = plsc.VectorSubcoreMesh(core_axis_name="c", subcore_axis_name="s")


@functools.partial(
    pl.kernel,
    out_type=jax.ShapeDtypeStruct((_BATCH, _HIDDEN), jnp.float32),
    mesh=_mesh,
    compiler_params=pltpu.CompilerParams(use_tc_tiling_on_sc=False),
    scratch_types=[
        pltpu.VMEM((_GROUPS, _G), jnp.int32),
        pltpu.VMEM((_GROUPS * _G, _HIDDEN), jnp.float32),
        pltpu.VMEM_SHARED((_NS * _WB, _HIDDEN), jnp.float32),
    ]
    + [pltpu.VMEM((_G,), jnp.int32) for _ in range(_ADD_GROUPS)]
    + [pltpu.SemaphoreType.DMA],
)
def _embed(table_hbm, idx_hbm, pos_hbm, out_hbm, idx_v, rows_v, acc_sh,
           p0, p1, p2, p3, p4, p5, sem):
    sid = lax.axis_index("s")
    wid = sid * 2 + lax.axis_index("c")
    pos_refs = (p0, p1, p2, p3, p4, p5)
    for j, p in enumerate(pos_refs):
        pltpu.sync_copy(pos_hbm.at[sid, j], p)
    for k in range(_NBLK):
        row = wid * _NBLK + k
        pltpu.sync_copy(idx_hbm.at[row], idx_v)
        copies = []
        for g in range(_GROUPS):
            dst = rows_v.at[pl.ds(g * _G, _G)]
            src = table_hbm.at[g // _C0_GROUPS].at[idx_v.at[g]]
            copies.append(pltpu.async_copy(src, dst, sem))
        for cpy in copies:
            cpy.wait()
        pltpu.sync_copy(rows_v.at[pl.ds(0, _WB)],
                        acc_sh.at[pl.ds(sid * _WB, _WB)])
        for j, p in enumerate(pos_refs):
            pltpu.sync_copy(rows_v.at[pl.ds(_WB + j * _G, _G)],
                            acc_sh.at[p], add=True)
        base = wid * _TOK_PER_W + k * _WB
        pltpu.sync_copy(acc_sh.at[pl.ds(sid * _WB, _WB)],
                        out_hbm.at[pl.ds(base, _WB)])


def kernel(indices, codebooks):
    table = codebooks
    flat = indices
    idx_arr = (flat.reshape(_NW, _NBLK, _WB, _NUM_CODEBOOKS)
               .transpose(0, 1, 3, 2)
               .reshape(_NW * _NBLK, _GROUPS, _G))
    sid_off = (jnp.arange(_NS, dtype=jnp.int32) * _WB)[:, None, None]
    grp_off = ((jnp.arange(_ADD_GROUPS, dtype=jnp.int32) % _C0_GROUPS)
               * _G)[None, :, None]
    lane = jnp.arange(_G, dtype=jnp.int32)[None, None, :]
    pos = sid_off + grp_off + lane
    return _embed(table, idx_arr, pos)
